# baseline (speedup 1.0000x reference)
import functools

import jax
import jax.numpy as jnp
from jax import lax
from jax.experimental import pallas as pl
from jax.experimental.pallas import tpu as pltpu
from jax.experimental.pallas import tpu_sc as plsc

NC = 2
NS = 16
NW = NC * NS
CHUNK = 128
NB = 4
DEG_K = 8


def _mesh():
    return plsc.VectorSubcoreMesh(core_axis_name="c", subcore_axis_name="s")


def _pad_rows(N):
    r = -(-N // NS)
    return -(-r // 8) * 8


def _deg_call(n_chunks, N):
    rows_per_tile = _pad_rows(N)
    NP = rows_per_tile * NS

    @functools.partial(
        pl.kernel,
        mesh=_mesh(),
        out_type=jax.ShapeDtypeStruct((NC, NP, 1), jnp.float32),
        compiler_params=pltpu.CompilerParams(use_tc_tiling_on_sc=False),
        scratch_types=[
            pltpu.VMEM((n_chunks, CHUNK), jnp.int32),
            pltpu.VMEM((CHUNK, 1), jnp.float32),
            pltpu.VMEM_SHARED((NP, 1), jnp.float32),
            pltpu.SemaphoreType.DMA((DEG_K,)),
        ],
    )
    def k(dsts_hbm, ones_hbm, zeros_hbm, out_hbm, dst_v, ones_v, acc, sems):
        c = lax.axis_index("c")
        s = lax.axis_index("s")
        wid = c * NS + s
        rbase = s * rows_per_tile
        pltpu.sync_copy(zeros_hbm, acc.at[pl.ds(rbase, rows_per_tile)])
        pltpu.sync_copy(dsts_hbm.at[wid], dst_v)
        pltpu.sync_copy(ones_hbm, ones_v)
        plsc.subcore_barrier()

        def outer(o, carry):
            for b in range(DEG_K):
                i = o * DEG_K + b
                pltpu.async_copy(ones_v, acc.at[dst_v.at[i]], sems.at[b],
                                 add=True)
            for b in range(DEG_K):
                i = o * DEG_K + b
                pltpu.make_async_copy(ones_v, acc.at[dst_v.at[i]],
                                      sems.at[b]).wait()
            return carry

        lax.fori_loop(0, n_chunks // DEG_K, outer, 0)
        plsc.subcore_barrier()
        pltpu.sync_copy(acc.at[pl.ds(rbase, rows_per_tile)],
                        out_hbm.at[c, pl.ds(rbase, rows_per_tile)])

    return k


def _agg_call(n_chunks, N, NGV, D):
    n_outer = n_chunks // NB
    rows_per_tile = _pad_rows(N)
    NP = rows_per_tile * NS

    @functools.partial(
        pl.kernel,
        mesh=_mesh(),
        out_type=jax.ShapeDtypeStruct((NC, NP, 128), jnp.float32),
        compiler_params=pltpu.CompilerParams(use_tc_tiling_on_sc=False),
        scratch_types=[
            pltpu.VMEM((n_chunks, CHUNK), jnp.int32),
            pltpu.VMEM((n_chunks, CHUNK), jnp.int32),
            pltpu.VMEM((NB, CHUNK, D), jnp.float32),
            pltpu.VMEM_SHARED((NP, D), jnp.float32),
            pltpu.SemaphoreType.DMA((NB,)),
            pltpu.SemaphoreType.DMA((NB,)),
        ],
    )
    def k(srcs_hbm, dsts_hbm, g_hbm, zeros_hbm, out_hbm,
          src_v, dst_v, rows_v, acc, sem_g, sem_s):
        c = lax.axis_index("c")
        s = lax.axis_index("s")
        wid = c * NS + s
        rbase = s * rows_per_tile
        pltpu.sync_copy(zeros_hbm, acc.at[pl.ds(rbase, rows_per_tile)])
        pltpu.sync_copy(srcs_hbm.at[wid], src_v)
        pltpu.sync_copy(dsts_hbm.at[wid], dst_v)
        plsc.subcore_barrier()

        def start_gather(i, b):
            pltpu.async_copy(g_hbm.at[src_v.at[i]], rows_v.at[b], sem_g.at[b])

        def wait_gather(i, b):
            pltpu.make_async_copy(g_hbm.at[src_v.at[i]], rows_v.at[b],
                                  sem_g.at[b]).wait()

        def start_scatter(i, b):
            pltpu.async_copy(rows_v.at[b], acc.at[dst_v.at[i]], sem_s.at[b],
                             add=True)

        def wait_scatter(i, b):
            pltpu.make_async_copy(rows_v.at[b], acc.at[dst_v.at[i]],
                                  sem_s.at[b]).wait()

        for b in range(NB):
            start_gather(b, b)

        def outer(o, carry):
            for b in range(NB):
                i = o * NB + b
                wait_gather(i, b)
                start_scatter(i, b)
                wait_scatter(i, b)
                start_gather(i + NB, b)
            return carry

        lax.fori_loop(0, n_outer - 1, outer, 0)
        for b in range(NB):
            i = (n_outer - 1) * NB + b
            wait_gather(i, b)
            start_scatter(i, b)
        for b in range(NB):
            i = (n_outer - 1) * NB + b
            wait_scatter(i, b)

        plsc.subcore_barrier()
        pltpu.sync_copy(acc.at[pl.ds(rbase, rows_per_tile)],
                        out_hbm.at[c, pl.ds(rbase, rows_per_tile),
                                   pl.ds(0, D)])

    return k


def _matmul(x, W):
    N, K = x.shape
    H = W.shape[1]
    R = 2000

    def body(x_ref, w_ref, h_ref):
        h_ref[...] = jnp.dot(x_ref[...], w_ref[...],
                             preferred_element_type=jnp.float32)

    return pl.pallas_call(
        body,
        grid=(N // R,),
        in_specs=[
            pl.BlockSpec((R, K), lambda i: (i, 0)),
            pl.BlockSpec((K, H), lambda i: (0, 0)),
        ],
        out_specs=pl.BlockSpec((R, H), lambda i: (i, 0)),
        out_shape=jax.ShapeDtypeStruct((N, H), jnp.float32),
    )(x, W)


def _scale_g(h, deg2d):
    N, H = h.shape
    R = 2560
    grid = -(-N // R)

    def body(d_ref, h_ref, g_ref, dinv_ref):
        dsum = d_ref[0:1, :] + d_ref[1:2, :]
        dinv_col = lax.rsqrt(dsum + 1.0).T
        g_ref[:, 0:H] = h_ref[...] * dinv_col
        g_ref[:, H:128] = jnp.zeros((R, 128 - H), jnp.float32)
        dinv_ref[...] = dinv_col

    return pl.pallas_call(
        body,
        grid=(grid,),
        in_specs=[
            pl.BlockSpec((NC, R), lambda i: (0, i)),
            pl.BlockSpec((R, H), lambda i: (i, 0)),
        ],
        out_specs=[
            pl.BlockSpec((R, 128), lambda i: (i, 0)),
            pl.BlockSpec((R, 1), lambda i: (i, 0)),
        ],
        out_shape=[
            jax.ShapeDtypeStruct((N, 128), jnp.float32),
            jax.ShapeDtypeStruct((N, 1), jnp.float32),
        ],
    )(deg2d, h)


def _mid_layer(accp, g1p, dinv, b1, W2):
    N = g1p.shape[0]
    H = b1.shape[1]
    O = W2.shape[1]
    R = 2000

    def body(a0_ref, a1_ref, g1_ref, dinv_ref, b1_ref, w2_ref, g2_ref):
        dinv = dinv_ref[...]
        a0 = a0_ref[0, :, 0:H]
        a1 = a1_ref[0, :, 0:H]
        g1 = g1_ref[:, 0:H]
        out1 = dinv * (a0 + a1 + g1) + b1_ref[...]
        out1 = jnp.maximum(out1, 0.0)
        g2_ref[:, 0:O] = jnp.dot(out1, w2_ref[...],
                                 preferred_element_type=jnp.float32) * dinv
        g2_ref[:, O:128] = jnp.zeros((R, 128 - O), jnp.float32)

    return pl.pallas_call(
        body,
        grid=(N // R,),
        in_specs=[
            pl.BlockSpec((1, R, 128), lambda i: (0, i, 0)),
            pl.BlockSpec((1, R, 128), lambda i: (1, i, 0)),
            pl.BlockSpec((R, 128), lambda i: (i, 0)),
            pl.BlockSpec((R, 1), lambda i: (i, 0)),
            pl.BlockSpec((1, H), lambda i: (0, 0)),
            pl.BlockSpec((H, O), lambda i: (0, 0)),
        ],
        out_specs=pl.BlockSpec((R, 128), lambda i: (i, 0)),
        out_shape=jax.ShapeDtypeStruct((N, 128), jnp.float32),
    )(accp, accp, g1p, dinv, b1, W2)


def _final_layer(accp, g2p, dinv, b2):
    N = g2p.shape[0]
    O = b2.shape[1]
    R = 2000

    def body(c0_ref, c1_ref, g2_ref, dinv_ref, b2_ref, o_ref):
        o_ref[...] = dinv_ref[...] * (c0_ref[0, :, 0:O] + c1_ref[0, :, 0:O]
                                      + g2_ref[:, 0:O]) + b2_ref[...]

    return pl.pallas_call(
        body,
        grid=(N // R,),
        in_specs=[
            pl.BlockSpec((1, R, 128), lambda i: (0, i, 0)),
            pl.BlockSpec((1, R, 128), lambda i: (1, i, 0)),
            pl.BlockSpec((R, 128), lambda i: (i, 0)),
            pl.BlockSpec((R, 1), lambda i: (i, 0)),
            pl.BlockSpec((1, O), lambda i: (0, 0)),
        ],
        out_specs=pl.BlockSpec((R, O), lambda i: (i, 0)),
        out_shape=jax.ShapeDtypeStruct((N, O), jnp.float32),
    )(accp, accp, g2p, dinv, b2)


def kernel(x, edge_index, W1, b1, W2, b2):
    N, _ = x.shape
    E = edge_index.shape[1]
    H = W1.shape[1]
    O = W2.shape[1]
    rows_per_tile = _pad_rows(N)
    NP = rows_per_tile * NS

    epw = -(-E // (NW * CHUNK)) * CHUNK
    n_chunks = epw // CHUNK
    pad_e = NW * epw - E
    src = jnp.concatenate([edge_index[0], jnp.zeros((pad_e,), jnp.int32)])
    dst = jnp.concatenate([edge_index[1],
                           jnp.full((pad_e,), NP - 1, jnp.int32)])
    dsts = dst.reshape(NW, n_chunks, CHUNK)
    srcs_h = (src * (128 // H)).reshape(NW, n_chunks, CHUNK)
    srcs_o = (src * (128 // O)).reshape(NW, n_chunks, CHUNK)

    ones_c = jnp.ones((CHUNK, 1), jnp.float32)
    zeros_deg = jnp.zeros((rows_per_tile, 1), jnp.float32)
    deg_parts = _deg_call(n_chunks, N)(dsts, ones_c, zeros_deg)
    deg2d = deg_parts.reshape(NC, NP)

    h1 = _matmul(x, W1)
    g1p, dinv = _scale_g(h1, deg2d)

    zeros_h = jnp.zeros((rows_per_tile, H), jnp.float32)
    g1v = g1p.reshape(N * (128 // H), H)
    acc1 = _agg_call(n_chunks, N, N * (128 // H), H)(srcs_h, dsts, g1v,
                                                     zeros_h)

    g2p = _mid_layer(acc1, g1p, dinv, b1.reshape(1, H), W2)

    zeros_o = jnp.zeros((rows_per_tile, O), jnp.float32)
    g2v = g2p.reshape(N * (128 // O), O)
    acc2 = _agg_call(n_chunks, N, N * (128 // O), O)(srcs_o, dsts, g2v,
                                                     zeros_o)

    return _final_layer(acc2, g2p, dinv, b2.reshape(1, O))


# speedup vs baseline: 1.5375x; 1.5375x over previous
import functools

import jax
import jax.numpy as jnp
from jax import lax
from jax.experimental import pallas as pl
from jax.experimental.pallas import tpu as pltpu
from jax.experimental.pallas import tpu_sc as plsc

NC = 2
NS = 16
NW = NC * NS
CHUNK = 128
NB = 4
DEG_K = 8


def _mesh():
    return plsc.VectorSubcoreMesh(core_axis_name="c", subcore_axis_name="s")


def _pad_rows(N):
    r = -(-N // NS)
    return -(-r // 8) * 8


def _deg_call(n_chunks, N):
    rows_per_tile = _pad_rows(N)
    NP = rows_per_tile * NS

    @functools.partial(
        pl.kernel,
        mesh=_mesh(),
        out_type=jax.ShapeDtypeStruct((NC, NP, 1), jnp.float32),
        compiler_params=pltpu.CompilerParams(use_tc_tiling_on_sc=False),
        scratch_types=[
            pltpu.VMEM((n_chunks, CHUNK), jnp.int32),
            pltpu.VMEM((CHUNK, 1), jnp.float32),
            pltpu.VMEM_SHARED((NP, 1), jnp.float32),
            pltpu.SemaphoreType.DMA((DEG_K,)),
        ],
    )
    def k(dsts_hbm, ones_hbm, zeros_hbm, out_hbm, dst_v, ones_v, acc, sems):
        c = lax.axis_index("c")
        s = lax.axis_index("s")
        wid = c * NS + s
        rbase = s * rows_per_tile
        pltpu.sync_copy(zeros_hbm, acc.at[pl.ds(rbase, rows_per_tile)])
        pltpu.sync_copy(dsts_hbm.at[wid], dst_v)
        pltpu.sync_copy(ones_hbm, ones_v)
        plsc.subcore_barrier()

        def outer(o, carry):
            for b in range(DEG_K):
                i = o * DEG_K + b
                pltpu.async_copy(ones_v, acc.at[dst_v.at[i]], sems.at[b],
                                 add=True)
            for b in range(DEG_K):
                i = o * DEG_K + b
                pltpu.make_async_copy(ones_v, acc.at[dst_v.at[i]],
                                      sems.at[b]).wait()
            return carry

        lax.fori_loop(0, n_chunks // DEG_K, outer, 0)
        plsc.subcore_barrier()
        pltpu.sync_copy(acc.at[pl.ds(rbase, rows_per_tile)],
                        out_hbm.at[c, pl.ds(rbase, rows_per_tile)])

    return k


def _agg_call(n_chunks, N, NGV, D):
    n_outer = n_chunks // NB
    rows_per_tile = _pad_rows(N)
    NP = rows_per_tile * NS

    @functools.partial(
        pl.kernel,
        mesh=_mesh(),
        out_type=jax.ShapeDtypeStruct((NC, NP, 128), jnp.float32),
        compiler_params=pltpu.CompilerParams(use_tc_tiling_on_sc=False),
        scratch_types=[
            pltpu.VMEM((n_chunks, CHUNK), jnp.int32),
            pltpu.VMEM((n_chunks, CHUNK), jnp.int32),
            pltpu.VMEM((NB, CHUNK, D), jnp.float32),
            pltpu.VMEM_SHARED((NP, D), jnp.float32),
            pltpu.SemaphoreType.DMA((NB,)),
            pltpu.SemaphoreType.DMA((NB,)),
        ],
    )
    def k(srcs_hbm, dsts_hbm, g_hbm, zeros_hbm, out_hbm,
          src_v, dst_v, rows_v, acc, sem_g, sem_s):
        c = lax.axis_index("c")
        s = lax.axis_index("s")
        wid = c * NS + s
        rbase = s * rows_per_tile
        pltpu.sync_copy(zeros_hbm, acc.at[pl.ds(rbase, rows_per_tile)])
        pltpu.sync_copy(srcs_hbm.at[wid], src_v)
        pltpu.sync_copy(dsts_hbm.at[wid], dst_v)
        plsc.subcore_barrier()

        def start_gather(i, b):
            pltpu.async_copy(g_hbm.at[src_v.at[i]], rows_v.at[b], sem_g.at[b])

        def wait_gather(i, b):
            pltpu.make_async_copy(g_hbm.at[src_v.at[i]], rows_v.at[b],
                                  sem_g.at[b]).wait()

        def start_scatter(i, b):
            pltpu.async_copy(rows_v.at[b], acc.at[dst_v.at[i]], sem_s.at[b],
                             add=True)

        def wait_scatter(i, b):
            pltpu.make_async_copy(rows_v.at[b], acc.at[dst_v.at[i]],
                                  sem_s.at[b]).wait()

        for b in range(NB):
            start_gather(b, b)

        def outer(o, carry):
            for b in range(NB):
                i = o * NB + b
                wait_gather(i, b)
                start_scatter(i, b)
                wait_scatter(i, b)
                start_gather(i + NB, b)
            return carry

        lax.fori_loop(0, n_outer - 1, outer, 0)
        for b in range(NB):
            i = (n_outer - 1) * NB + b
            wait_gather(i, b)
            start_scatter(i, b)
        for b in range(NB):
            i = (n_outer - 1) * NB + b
            wait_scatter(i, b)

        plsc.subcore_barrier()
        pltpu.sync_copy(acc.at[pl.ds(rbase, rows_per_tile)],
                        out_hbm.at[c, pl.ds(rbase, rows_per_tile),
                                   pl.ds(0, D)])

    return k


def _matmul(x, W):
    N, K = x.shape
    H = W.shape[1]
    R = 2000

    def body(x_ref, w_ref, h_ref):
        h_ref[...] = jnp.dot(x_ref[...], w_ref[...],
                             preferred_element_type=jnp.float32)

    return pl.pallas_call(
        body,
        grid=(N // R,),
        in_specs=[
            pl.BlockSpec((R, K), lambda i: (i, 0)),
            pl.BlockSpec((K, H), lambda i: (0, 0)),
        ],
        out_specs=pl.BlockSpec((R, H), lambda i: (i, 0)),
        out_shape=jax.ShapeDtypeStruct((N, H), jnp.float32),
    )(x, W)


def _scale_g(h, deg2d):
    N, H = h.shape
    R = 2560
    grid = -(-N // R)

    def body(d_ref, h_ref, g_ref, dinv_ref):
        dsum = d_ref[0:1, :] + d_ref[1:2, :]
        dinv_col = lax.rsqrt(dsum + 1.0).T
        g_ref[:, 0:H] = h_ref[...] * dinv_col
        g_ref[:, H:128] = jnp.zeros((R, 128 - H), jnp.float32)
        dinv_ref[...] = dinv_col

    return pl.pallas_call(
        body,
        grid=(grid,),
        in_specs=[
            pl.BlockSpec((NC, R), lambda i: (0, i)),
            pl.BlockSpec((R, H), lambda i: (i, 0)),
        ],
        out_specs=[
            pl.BlockSpec((R, 128), lambda i: (i, 0)),
            pl.BlockSpec((R, 1), lambda i: (i, 0)),
        ],
        out_shape=[
            jax.ShapeDtypeStruct((N, 128), jnp.float32),
            jax.ShapeDtypeStruct((N, 1), jnp.float32),
        ],
    )(deg2d, h)


def _mid_layer(accp, g1p, dinv, b1, W2):
    N = g1p.shape[0]
    H = b1.shape[1]
    O = W2.shape[1]
    R = 2000

    def body(a0_ref, a1_ref, g1_ref, dinv_ref, b1_ref, w2_ref, g2_ref):
        dinv = dinv_ref[...]
        a0 = a0_ref[0, :, 0:H]
        a1 = a1_ref[0, :, 0:H]
        g1 = g1_ref[:, 0:H]
        out1 = dinv * (a0 + a1 + g1) + b1_ref[...]
        out1 = jnp.maximum(out1, 0.0)
        g2_ref[:, 0:O] = jnp.dot(out1, w2_ref[...],
                                 preferred_element_type=jnp.float32) * dinv
        g2_ref[:, O:128] = jnp.zeros((R, 128 - O), jnp.float32)

    return pl.pallas_call(
        body,
        grid=(N // R,),
        in_specs=[
            pl.BlockSpec((1, R, 128), lambda i: (0, i, 0)),
            pl.BlockSpec((1, R, 128), lambda i: (1, i, 0)),
            pl.BlockSpec((R, 128), lambda i: (i, 0)),
            pl.BlockSpec((R, 1), lambda i: (i, 0)),
            pl.BlockSpec((1, H), lambda i: (0, 0)),
            pl.BlockSpec((H, O), lambda i: (0, 0)),
        ],
        out_specs=pl.BlockSpec((R, 128), lambda i: (i, 0)),
        out_shape=jax.ShapeDtypeStruct((N, 128), jnp.float32),
    )(accp, accp, g1p, dinv, b1, W2)


def _final_layer(accp, g2p, dinv, b2):
    N = g2p.shape[0]
    O = b2.shape[1]
    R = 2000

    def body(c0_ref, c1_ref, g2_ref, dinv_ref, b2_ref, o_ref):
        o_ref[...] = dinv_ref[...] * (c0_ref[0, :, 0:O] + c1_ref[0, :, 0:O]
                                      + g2_ref[:, 0:O]) + b2_ref[...]

    return pl.pallas_call(
        body,
        grid=(N // R,),
        in_specs=[
            pl.BlockSpec((1, R, 128), lambda i: (0, i, 0)),
            pl.BlockSpec((1, R, 128), lambda i: (1, i, 0)),
            pl.BlockSpec((R, 128), lambda i: (i, 0)),
            pl.BlockSpec((R, 1), lambda i: (i, 0)),
            pl.BlockSpec((1, O), lambda i: (0, 0)),
        ],
        out_specs=pl.BlockSpec((R, O), lambda i: (i, 0)),
        out_shape=jax.ShapeDtypeStruct((N, O), jnp.float32),
    )(accp, accp, g2p, dinv, b2)


def kernel(x, edge_index, W1, b1, W2, b2):
    N, _ = x.shape
    E = edge_index.shape[1]
    H = W1.shape[1]
    O = W2.shape[1]
    rows_per_tile = _pad_rows(N)
    NP = rows_per_tile * NS

    epw = -(-E // (NW * CHUNK)) * CHUNK
    n_chunks = epw // CHUNK
    pad_e = NW * epw - E
    pad_i = jnp.arange(pad_e, dtype=jnp.int32)
    src = jnp.concatenate([edge_index[0], pad_i % N])
    dst = jnp.concatenate([edge_index[1], N + pad_i % (NP - N)])
    dsts = dst.reshape(NW, n_chunks, CHUNK)
    srcs_h = (src * (128 // H)).reshape(NW, n_chunks, CHUNK)
    srcs_o = (src * (128 // O)).reshape(NW, n_chunks, CHUNK)

    ones_c = jnp.ones((CHUNK, 1), jnp.float32)
    zeros_deg = jnp.zeros((rows_per_tile, 1), jnp.float32)
    deg_parts = _deg_call(n_chunks, N)(dsts, ones_c, zeros_deg)
    deg2d = deg_parts.reshape(NC, NP)

    h1 = _matmul(x, W1)
    g1p, dinv = _scale_g(h1, deg2d)

    zeros_h = jnp.zeros((rows_per_tile, H), jnp.float32)
    g1v = g1p.reshape(N * (128 // H), H)
    acc1 = _agg_call(n_chunks, N, N * (128 // H), H)(srcs_h, dsts, g1v,
                                                     zeros_h)

    g2p = _mid_layer(acc1, g1p, dinv, b1.reshape(1, H), W2)

    zeros_o = jnp.zeros((rows_per_tile, O), jnp.float32)
    g2v = g2p.reshape(N * (128 // O), O)
    acc2 = _agg_call(n_chunks, N, N * (128 // O), O)(srcs_o, dsts, g2v,
                                                     zeros_o)

    return _final_layer(acc2, g2p, dinv, b2.reshape(1, O))
